# baseline (speedup 1.0000x reference)
import functools
import numpy as np
import jax
import jax.numpy as jnp
from jax import lax
from jax.experimental import pallas as pl
from jax.experimental.pallas import tpu as pltpu
from jax.experimental.pallas import tpu_sc as plsc

_H = 16
_OLD = 2 * 24 - 1
_NEW = 2 * 32 - 1
_S = 32 * 32 + 1
_WPH = 60 * 32
_NW = 32


def _keys_cubic(x):
    out = ((1.5 * x - 2.5) * x) * x + 1.0
    out = np.where(x >= 1.0, ((-0.5 * x + 2.5) * x - 4.0) * x + 2.0, out)
    return np.where(x >= 2.0, 0.0, out)


def _weight_mat(in_size, out_size):
    scale = out_size / in_size
    sample_f = (np.arange(out_size, dtype=np.float64) + 0.5) / scale - 0.5
    x = np.abs(sample_f[None, :] - np.arange(in_size, dtype=np.float64)[:, None])
    w = _keys_cubic(x)
    total = w.sum(axis=0, keepdims=True)
    w = np.where(np.abs(total) > 1000.0 * float(np.finfo(np.float32).eps),
                 w / np.where(total != 0, total, 1.0), 0.0)
    ok = (sample_f >= -0.5) & (sample_f <= in_size - 0.5)
    return np.where(ok[None, :], w, 0.0)


_WMAT = _weight_mat(_OLD, _NEW)
_RH = np.ascontiguousarray(_WMAT.T).astype(np.float32)
_RWREV = np.zeros((_OLD, 128), np.float32)
_RWREV[:, :_NEW] = _WMAT[:, ::-1]


def _w128_row(h, i):
    p = i - 1
    ph, pw = divmod(p, 32)
    return [(h * 60 + (ph - 4 * u + 28)) * 32 + pw for u in range(8)]


def _build_gidx():
    g = np.zeros((1024, 128), np.int32)
    for gg in range(1024):
        h, kk = divmod(gg, 64)
        for r in range(16):
            i = 16 * kk + r
            if i >= 1:
                g[gg, r * 8: r * 8 + 8] = _w128_row(h, i)
    return g


_GIDX = _build_gidx()
_TIDX = np.array([_w128_row(h, 1024) for h in range(_H)], np.int32)


def _tc_body(delta_ref, brd_ref, old_ref, rh_ref, rw_ref,
             w_ref, row_ref, col_ref):
    d = delta_ref[0, 0]
    h = pl.program_id(0)
    old = old_ref[0]
    imgrev = jnp.dot(rh_ref[...],
                     jnp.dot(old, rw_ref[...], preferred_element_type=jnp.float32),
                     preferred_element_type=jnp.float32) + d
    pieces = []
    for j in range(4):
        base = jax.lax.slice(imgrev, (3 - j, 0), (63 - j, 128))
        bexp = jnp.broadcast_to(base[:, None, :], (60, 32, 128))
        rolled = pltpu.roll(bexp, 97, 2, stride=1, stride_axis=1)
        pieces.append(rolled[:, :, :32])
    cc = jnp.concatenate(pieces, axis=2)
    w_ref[...] = cc.reshape(_WPH, 128)
    v1 = brd_ref[h, 0] + d
    v2 = brd_ref[h, 1] + d
    v3 = brd_ref[h, 2] + d
    col = jax.lax.broadcasted_iota(jnp.int32, (1, _S), 1)
    row_ref[...] = jnp.where(col == 0, v3, v1)[None]
    col_ref[...] = jnp.full((1, 1, 128), v2, jnp.float32)


_GDN = lax.GatherDimensionNumbers(offset_dims=(), collapsed_slice_dims=(0,),
                                  start_index_map=(0,))


def _take(v, perm):
    return lax.gather(v, perm.reshape(16, 1), _GDN, slice_sizes=(1,),
                      mode=lax.GatherScatterMode.PROMISE_IN_BOUNDS)


def _sc_expand_body(w_hbm, gidx_hbm, row_hbm, v2_hbm, tidx_hbm, out_hbm,
                    idxv, stag, buf, colc, tidxv, tstag, gsem):
    c = lax.axis_index("c")
    s = lax.axis_index("s")
    wid = s * 2 + c
    lane = lax.iota(jnp.int32, 16)
    rot1 = lax.rem(lane + 15, 16)

    def repack_row(src_ref, base, r, v2v):
        for q in range(63, -1, -1):
            buf[r, pl.ds(1 + 16 * q, 16)] = src_ref[base + q // 8,
                                                    pl.ds((q % 8) * 16, 16)]
        cur0 = src_ref[base, pl.ds(0, 16)]
        rowstart = _take(jnp.where(lane == 15, v2v, cur0), rot1)
        buf[r, pl.ds(0, 16)] = rowstart

    def body(k, carry):
        g = wid + _NW * k
        h = g // 64
        kk = lax.rem(g, 64)
        i0 = 16 * kk
        pltpu.sync_copy(gidx_hbm.at[g], idxv)
        pltpu.async_copy(w_hbm.at[idxv], stag, gsem).wait()
        pltpu.sync_copy(v2_hbm.at[h, 0], colc)
        v2v = colc[pl.ds(0, 16)]
        for r in range(16):
            repack_row(stag, 8 * r, r, v2v)

        @pl.when(kk == 0)
        def _():
            pltpu.sync_copy(row_hbm.at[h, 0], buf.at[0])

        view = out_hbm.at[h, pl.ds(pl.multiple_of(i0, 8), 16)]
        pltpu.sync_copy(buf, view)
        return carry

    lax.fori_loop(0, 32, body, 0)

    @pl.when(wid < _H)
    def _():
        h = wid
        pltpu.sync_copy(tidx_hbm.at[h], tidxv)
        pltpu.async_copy(w_hbm.at[tidxv], tstag, gsem).wait()
        pltpu.sync_copy(v2_hbm.at[h, 0], colc)
        v2v = colc[pl.ds(0, 16)]
        repack_row(tstag, 0, 0, v2v)
        pltpu.sync_copy(buf.at[pl.ds(0, 1)], out_hbm.at[h, pl.ds(1024, 1)])


_SC_KERNEL_CACHE = []


def _sc_expand():
    if not _SC_KERNEL_CACHE:
        mesh = plsc.VectorSubcoreMesh(core_axis_name="c", subcore_axis_name="s")
        k = functools.partial(
            pl.kernel, mesh=mesh,
            out_type=jax.ShapeDtypeStruct((_H, _S, _S), jnp.float32),
            scratch_types=[
                pltpu.VMEM((128,), jnp.int32),
                pltpu.VMEM((128, 128), jnp.float32),
                pltpu.VMEM((16, _S), jnp.float32),
                pltpu.VMEM((128,), jnp.float32),
                pltpu.VMEM((8,), jnp.int32),
                pltpu.VMEM((8, 128), jnp.float32),
                pltpu.SemaphoreType.DMA,
            ],
        )(_sc_expand_body)
        _SC_KERNEL_CACHE.append(k)
    return _SC_KERNEL_CACHE[0]


def kernel(relative_position_bias_table, training_window_size):
    tab = relative_position_bias_table
    tws = training_window_size
    delta = jnp.sum(tws - jnp.asarray((32, 32), dtype=tws.dtype)).astype(tab.dtype)
    delta2 = jnp.reshape(delta, (1, 1))
    old = tab[:-3, :].T.reshape(_H, _OLD, _OLD)
    brd = tab[-3:, :].T
    w128, rowtab, v2tab = pl.pallas_call(
        _tc_body,
        grid=(_H,),
        in_specs=[
            pl.BlockSpec(memory_space=pltpu.SMEM),
            pl.BlockSpec(memory_space=pltpu.SMEM),
            pl.BlockSpec((1, _OLD, _OLD), lambda h: (h, 0, 0)),
            pl.BlockSpec((_NEW, _OLD), lambda h: (0, 0)),
            pl.BlockSpec((_OLD, 128), lambda h: (0, 0)),
        ],
        out_specs=[
            pl.BlockSpec((_WPH, 128), lambda h: (h, 0)),
            pl.BlockSpec((1, 1, _S), lambda h: (h, 0, 0)),
            pl.BlockSpec((1, 1, 128), lambda h: (h, 0, 0)),
        ],
        out_shape=[
            jax.ShapeDtypeStruct((_H * _WPH, 128), jnp.float32),
            jax.ShapeDtypeStruct((_H, 1, _S), jnp.float32),
            jax.ShapeDtypeStruct((_H, 1, 128), jnp.float32),
        ],
    )(delta2, brd, old, jnp.asarray(_RH), jnp.asarray(_RWREV))
    return _sc_expand()(w128, jnp.asarray(_GIDX), rowtab, v2tab,
                        jnp.asarray(_TIDX))


# speedup vs baseline: 1.0717x; 1.0717x over previous
import functools
import numpy as np
import jax
import jax.numpy as jnp
from jax import lax
from jax.experimental import pallas as pl
from jax.experimental.pallas import tpu as pltpu
from jax.experimental.pallas import tpu_sc as plsc

_H = 16
_OLD = 2 * 24 - 1
_NEW = 2 * 32 - 1
_S = 32 * 32 + 1
_WPH = 60 * 32
_NW = 32


def _keys_cubic(x):
    out = ((1.5 * x - 2.5) * x) * x + 1.0
    out = np.where(x >= 1.0, ((-0.5 * x + 2.5) * x - 4.0) * x + 2.0, out)
    return np.where(x >= 2.0, 0.0, out)


def _weight_mat(in_size, out_size):
    scale = out_size / in_size
    sample_f = (np.arange(out_size, dtype=np.float64) + 0.5) / scale - 0.5
    x = np.abs(sample_f[None, :] - np.arange(in_size, dtype=np.float64)[:, None])
    w = _keys_cubic(x)
    total = w.sum(axis=0, keepdims=True)
    w = np.where(np.abs(total) > 1000.0 * float(np.finfo(np.float32).eps),
                 w / np.where(total != 0, total, 1.0), 0.0)
    ok = (sample_f >= -0.5) & (sample_f <= in_size - 0.5)
    return np.where(ok[None, :], w, 0.0)


_WMAT = _weight_mat(_OLD, _NEW)
_RH = np.ascontiguousarray(_WMAT.T).astype(np.float32)
_RWREV = np.zeros((_OLD, 128), np.float32)
_RWREV[:, :_NEW] = _WMAT[:, ::-1]


def _w128_row(h, i):
    p = i - 1
    ph, pw = divmod(p, 32)
    return [(h * 60 + (ph - 4 * u + 28)) * 32 + pw for u in range(8)]


def _build_gidx():
    g = np.zeros((1024, 128), np.int32)
    for gg in range(1024):
        h, kk = divmod(gg, 64)
        for r in range(16):
            i = 16 * kk + r
            if i >= 1:
                g[gg, r * 8: r * 8 + 8] = _w128_row(h, i)
    return g


_GIDX = _build_gidx()
_TIDX = np.array([_w128_row(h, 1024) for h in range(_H)], np.int32)


def _tc_body(delta_ref, brd_ref, old_ref, rh_ref, rw_ref,
             w_ref, row_ref, col_ref):
    d = delta_ref[0, 0]
    h = pl.program_id(0)
    old = old_ref[0]
    imgrev = jnp.dot(rh_ref[...],
                     jnp.dot(old, rw_ref[...], preferred_element_type=jnp.float32),
                     preferred_element_type=jnp.float32) + d
    pieces = []
    for j in range(4):
        base = jax.lax.slice(imgrev, (3 - j, 0), (63 - j, 128))
        bexp = jnp.broadcast_to(base[:, None, :], (60, 32, 128))
        rolled = pltpu.roll(bexp, 97, 2, stride=1, stride_axis=1)
        pieces.append(rolled[:, :, :32])
    cc = jnp.concatenate(pieces, axis=2)
    w_ref[...] = cc.reshape(_WPH, 128)
    v1 = brd_ref[h, 0] + d
    v2 = brd_ref[h, 1] + d
    v3 = brd_ref[h, 2] + d
    col = jax.lax.broadcasted_iota(jnp.int32, (1, _S), 1)
    row_ref[...] = jnp.where(col == 0, v3, v1)[None]
    col_ref[...] = jnp.full((1, 1, 128), v2, jnp.float32)


_GDN = lax.GatherDimensionNumbers(offset_dims=(), collapsed_slice_dims=(0,),
                                  start_index_map=(0,))


def _take(v, perm):
    return lax.gather(v, perm.reshape(16, 1), _GDN, slice_sizes=(1,),
                      mode=lax.GatherScatterMode.PROMISE_IN_BOUNDS)


def _sc_expand_body(w_hbm, gidx_hbm, row_hbm, v2_hbm, tidx_hbm, out_hbm,
                    idxv, stag, buf, v2allv, tidxv, tstag, gsem, wsem):
    c = lax.axis_index("c")
    s = lax.axis_index("s")
    wid = s * 2 + c
    lane = lax.iota(jnp.int32, 16)
    rot1 = lax.rem(lane + 15, 16)
    pltpu.sync_copy(v2_hbm, v2allv)

    def repack_row(src_ref, base, r, v2v):
        for q in range(63, -1, -1):
            buf[r, pl.ds(1 + 16 * q, 16)] = src_ref[base + q // 8,
                                                    pl.ds((q % 8) * 16, 16)]
        cur0 = src_ref[base, pl.ds(0, 16)]
        rowstart = _take(jnp.where(lane == 15, v2v, cur0), rot1)
        buf[r, pl.ds(0, 16)] = rowstart

    def body(k, carry):
        g = wid + _NW * k
        h = g // 64
        kk = lax.rem(g, 64)
        i0 = 16 * kk
        pltpu.sync_copy(gidx_hbm.at[g], idxv)
        pltpu.async_copy(w_hbm.at[idxv], stag, gsem)
        v2v = v2allv[h, 0, pl.ds(0, 16)]

        @pl.when(k >= 1)
        def _():
            pltpu.make_async_copy(out_hbm.at[0, pl.ds(0, 16)], buf, wsem).wait()

        pltpu.make_async_copy(w_hbm.at[pl.ds(0, 128)], stag, gsem).wait()
        for r in range(16):
            repack_row(stag, 8 * r, r, v2v)

        @pl.when(kk == 0)
        def _():
            pltpu.sync_copy(row_hbm.at[h, 0], buf.at[0])

        view = out_hbm.at[h, pl.ds(pl.multiple_of(i0, 8), 16)]
        pltpu.async_copy(buf, view, wsem)
        return carry

    lax.fori_loop(0, 32, body, 0)
    pltpu.make_async_copy(out_hbm.at[0, pl.ds(0, 16)], buf, wsem).wait()

    @pl.when(wid < _H)
    def _():
        h = wid
        pltpu.sync_copy(tidx_hbm.at[h], tidxv)
        pltpu.async_copy(w_hbm.at[tidxv], tstag, gsem).wait()
        v2v = v2allv[h, 0, pl.ds(0, 16)]
        repack_row(tstag, 0, 0, v2v)
        pltpu.sync_copy(buf.at[pl.ds(0, 1)], out_hbm.at[h, pl.ds(1024, 1)])


_SC_KERNEL_CACHE = []


def _sc_expand():
    if not _SC_KERNEL_CACHE:
        mesh = plsc.VectorSubcoreMesh(core_axis_name="c", subcore_axis_name="s")
        k = functools.partial(
            pl.kernel, mesh=mesh,
            out_type=jax.ShapeDtypeStruct((_H, _S, _S), jnp.float32),
            scratch_types=[
                pltpu.VMEM((128,), jnp.int32),
                pltpu.VMEM((128, 128), jnp.float32),
                pltpu.VMEM((16, _S), jnp.float32),
                pltpu.VMEM((_H, 1, 128), jnp.float32),
                pltpu.VMEM((8,), jnp.int32),
                pltpu.VMEM((8, 128), jnp.float32),
                pltpu.SemaphoreType.DMA,
                pltpu.SemaphoreType.DMA,
            ],
        )(_sc_expand_body)
        _SC_KERNEL_CACHE.append(k)
    return _SC_KERNEL_CACHE[0]


def kernel(relative_position_bias_table, training_window_size):
    tab = relative_position_bias_table
    tws = training_window_size
    delta = jnp.sum(tws - jnp.asarray((32, 32), dtype=tws.dtype)).astype(tab.dtype)
    delta2 = jnp.reshape(delta, (1, 1))
    old = tab[:-3, :].T.reshape(_H, _OLD, _OLD)
    brd = tab[-3:, :].T
    w128, rowtab, v2tab = pl.pallas_call(
        _tc_body,
        grid=(_H,),
        in_specs=[
            pl.BlockSpec(memory_space=pltpu.SMEM),
            pl.BlockSpec(memory_space=pltpu.SMEM),
            pl.BlockSpec((1, _OLD, _OLD), lambda h: (h, 0, 0)),
            pl.BlockSpec((_NEW, _OLD), lambda h: (0, 0)),
            pl.BlockSpec((_OLD, 128), lambda h: (0, 0)),
        ],
        out_specs=[
            pl.BlockSpec((_WPH, 128), lambda h: (h, 0)),
            pl.BlockSpec((1, 1, _S), lambda h: (h, 0, 0)),
            pl.BlockSpec((1, 1, 128), lambda h: (h, 0, 0)),
        ],
        out_shape=[
            jax.ShapeDtypeStruct((_H * _WPH, 128), jnp.float32),
            jax.ShapeDtypeStruct((_H, 1, _S), jnp.float32),
            jax.ShapeDtypeStruct((_H, 1, 128), jnp.float32),
        ],
    )(delta2, brd, old, jnp.asarray(_RH), jnp.asarray(_RWREV))
    return _sc_expand()(w128, jnp.asarray(_GIDX), rowtab, v2tab,
                        jnp.asarray(_TIDX))
